# baseline (speedup 1.0000x reference)
import functools

import jax
import jax.numpy as jnp
from jax import lax
from jax.experimental import pallas as pl
from jax.experimental.pallas import tpu as pltpu
from jax.experimental.pallas import tpu_sc as plsc

_VOCAB = 100000
_DIM = 128
_OUT = 32
_B = 4096
_L = 200

_NC = 2
_NS = 16
_NW = _NC * _NS
_IPT = _B // _NW
_LANES = 16


def _bf16_bits(x):
    u = lax.bitcast_convert_type(x, jnp.int32)
    return (u + jnp.int32(0x7FFF) + ((u >> 16) & 1)) >> 16


_PACK = _DIM // _OUT
_VS = _VOCAB // _PACK


def _project_body(t0, t1, t2, t3, w_ref, o_ref):
    dn = (((1,), (1,)), ((), ()))
    for j, tr in enumerate((t0, t1, t2, t3)):
        o_ref[:, _OUT * j:_OUT * (j + 1)] = lax.dot_general(
            tr[...], w_ref[...], dimension_numbers=dn,
            preferred_element_type=jnp.float32)


def _project_table(table, W):
    rows_blk = 5000
    grid = _VS // rows_blk
    in_specs = [
        pl.BlockSpec((rows_blk, _DIM),
                     (lambda g, jj=j: (g + jj * grid, 0)))
        for j in range(_PACK)
    ]
    in_specs.append(pl.BlockSpec((_OUT, _DIM), lambda g: (0, 0)))
    return pl.pallas_call(
        _project_body,
        grid=(grid,),
        in_specs=in_specs,
        out_specs=pl.BlockSpec((rows_blk, _DIM), lambda g: (g, 0)),
        out_shape=jax.ShapeDtypeStruct((_VS, _DIM), jnp.float32),
    )(table, table, table, table, W)


_NBUF = 4
_LPG = 5


def _pool_body(xr_hbm, tp_hbm, b_hbm, out_hbm,
               idx_v, bufs, acc, bias_v, sems):
    wid = lax.axis_index("s") * _NC + lax.axis_index("c")
    pltpu.sync_copy(xr_hbm.at[wid], idx_v)
    pltpu.sync_copy(b_hbm, bias_v)

    zero = jnp.zeros((_LANES,), jnp.float32)

    def zr(r, c):
        acc[r, pl.ds(0, _LANES)] = zero
        acc[r, pl.ds(_LANES, _LANES)] = zero
        return c
    lax.fori_loop(0, _IPT, zr, 0, unroll=8)

    nsteps = _L // _LPG

    for j in range(_NBUF):
        pltpu.make_async_copy(
            tp_hbm.at[idx_v.at[pl.ds(j * _LPG * _IPT, _LPG * _IPT)]],
            bufs[j], sems[j]).start()

    def outer(i, c):
        s0 = i * _NBUF
        for j in range(_NBUF):
            s = s0 + j
            buf = bufs[j]
            sem = sems[j]
            pltpu.make_async_copy(
                tp_hbm.at[idx_v.at[pl.ds(s * _LPG * _IPT, _LPG * _IPT)]],
                buf, sem).wait()

            def accum(r, cc):
                v0 = buf[r, pl.ds(0, _LANES)]
                v1 = buf[r, pl.ds(_LANES, _LANES)]
                for part in range(1, _LPG):
                    rr = part * _IPT + r
                    v0 = v0 + buf[rr, pl.ds(0, _LANES)]
                    v1 = v1 + buf[rr, pl.ds(_LANES, _LANES)]
                plsc.addupdate(acc.at[r, pl.ds(0, _LANES)], v0)
                plsc.addupdate(acc.at[r, pl.ds(_LANES, _LANES)], v1)
                return cc
            lax.fori_loop(0, _IPT, accum, 0, unroll=16)

            snext = s + _NBUF

            @pl.when(snext < nsteps)
            def _():
                pltpu.make_async_copy(
                    tp_hbm.at[idx_v.at[pl.ds(snext * _LPG * _IPT,
                                             _LPG * _IPT)]], buf, sem
                ).start()
        return c
    lax.fori_loop(0, nsteps // _NBUF, outer, 0)

    scale = jnp.float32(1.0 / _L)
    blo = bias_v[pl.ds(0, _LANES)]
    bhi = bias_v[pl.ds(_LANES, _LANES)]

    def fin(r, c):
        v0 = acc[r, pl.ds(0, _LANES)] * scale + blo
        v1 = acc[r, pl.ds(_LANES, _LANES)] * scale + bhi
        acc[r, pl.ds(0, _LANES)] = 1.0 / (1.0 + jnp.exp(-v0))
        acc[r, pl.ds(_LANES, _LANES)] = 1.0 / (1.0 + jnp.exp(-v1))
        return c
    lax.fori_loop(0, _IPT, fin, 0, unroll=4)

    pltpu.sync_copy(acc, out_hbm.at[pl.ds(wid * _IPT, _IPT), :])


@functools.partial(
    pl.kernel,
    mesh=plsc.VectorSubcoreMesh(core_axis_name="c", subcore_axis_name="s"),
    compiler_params=pltpu.CompilerParams(
        use_tc_tiling_on_sc=False, needs_layout_passes=False),
    out_type=jax.ShapeDtypeStruct((_B, _OUT), jnp.float32),
    scratch_types=[
        pltpu.VMEM((_L * _IPT,), jnp.int32),
        [pltpu.VMEM((_LPG * _IPT, _OUT), jnp.float32) for _ in range(_NBUF)],
        pltpu.VMEM((_IPT, _OUT), jnp.float32),
        pltpu.VMEM((_OUT,), jnp.float32),
        [pltpu.SemaphoreType.DMA for _ in range(_NBUF)],
    ],
)
def _pool(xr_hbm, tp_hbm, b_hbm, out_hbm,
          idx_v, bufs, acc, bias_v, sems):
    _pool_body(xr_hbm, tp_hbm, b_hbm, out_hbm,
               idx_v, bufs, acc, bias_v, sems)


def kernel(x, table, W, b):
    x = x.astype(jnp.int32)
    tp = _project_table(table, W).reshape(_VOCAB, _OUT)
    xq = (x % _VS) * _PACK + x // _VS
    xr = xq.reshape(_NW, _IPT, _L).transpose(0, 2, 1).reshape(_NW, _L * _IPT)
    return _pool(xr, tp, b)


# speedup vs baseline: 1.0075x; 1.0075x over previous
import functools

import jax
import jax.numpy as jnp
from jax import lax
from jax.experimental import pallas as pl
from jax.experimental.pallas import tpu as pltpu
from jax.experimental.pallas import tpu_sc as plsc

_VOCAB = 100000
_DIM = 128
_OUT = 32
_B = 4096
_L = 200

_NC = 2
_NS = 16
_NW = _NC * _NS
_IPT = _B // _NW
_LANES = 16


def _bf16_bits(x):
    u = lax.bitcast_convert_type(x, jnp.int32)
    return (u + jnp.int32(0x7FFF) + ((u >> 16) & 1)) >> 16


_PACK = _DIM // _OUT
_VS = _VOCAB // _PACK


def _project_body(t0, t1, t2, t3, w_ref, o_ref):
    dn = (((1,), (1,)), ((), ()))
    for j, tr in enumerate((t0, t1, t2, t3)):
        o_ref[:, _OUT * j:_OUT * (j + 1)] = lax.dot_general(
            tr[...], w_ref[...], dimension_numbers=dn,
            preferred_element_type=jnp.float32)


def _project_table(table, W):
    rows_blk = 5000
    grid = _VS // rows_blk
    in_specs = [
        pl.BlockSpec((rows_blk, _DIM),
                     (lambda g, jj=j: (g + jj * grid, 0)))
        for j in range(_PACK)
    ]
    in_specs.append(pl.BlockSpec((_OUT, _DIM), lambda g: (0, 0)))
    return pl.pallas_call(
        _project_body,
        grid=(grid,),
        in_specs=in_specs,
        out_specs=pl.BlockSpec((rows_blk, _DIM), lambda g: (g, 0)),
        out_shape=jax.ShapeDtypeStruct((_VS, _DIM), jnp.float32),
    )(table, table, table, table, W)


_NBUF = 3
_RPS = 4


def _pool_body(xr_hbm, tp_hbm, b_hbm, out_hbm,
               idx_v, bufs, acc, bias_v, sems):
    wid = lax.axis_index("s") * _NC + lax.axis_index("c")
    pltpu.sync_copy(xr_hbm.at[wid], idx_v)
    pltpu.sync_copy(b_hbm, bias_v)

    span = _RPS * _L
    nsteps = _IPT // _RPS
    ntail = nsteps % _NBUF

    def _start(s, buf, sem):
        pltpu.make_async_copy(
            tp_hbm.at[idx_v.at[pl.ds(s * span, span)]], buf, sem).start()

    def _wait(buf, sem):
        pltpu.make_async_copy(
            tp_hbm.at[idx_v.at[pl.ds(0, span)]], buf, sem).wait()

    for j in range(_NBUF):
        _start(j, bufs[j], sems[j])

    scale = jnp.float32(1.0 / _L)
    blo = bias_v[pl.ds(0, _LANES)]
    bhi = bias_v[pl.ds(_LANES, _LANES)]
    zero = jnp.zeros((_LANES,), jnp.float32)

    def _step(s, buf):
        for rr in range(_RPS):
            rb = rr * _L

            def red(k, carry):
                a0, a1, c0, c1 = carry
                base = rb + k * 4
                a0 = a0 + buf[base, pl.ds(0, _LANES)]
                a1 = a1 + buf[base, pl.ds(_LANES, _LANES)]
                c0 = c0 + buf[base + 1, pl.ds(0, _LANES)]
                c1 = c1 + buf[base + 1, pl.ds(_LANES, _LANES)]
                a0 = a0 + buf[base + 2, pl.ds(0, _LANES)]
                a1 = a1 + buf[base + 2, pl.ds(_LANES, _LANES)]
                c0 = c0 + buf[base + 3, pl.ds(0, _LANES)]
                c1 = c1 + buf[base + 3, pl.ds(_LANES, _LANES)]
                return (a0, a1, c0, c1)

            a0, a1, c0, c1 = lax.fori_loop(
                0, _L // 4, red, (zero, zero, zero, zero), unroll=4)
            v0 = (a0 + c0) * scale + blo
            v1 = (a1 + c1) * scale + bhi
            r = s * _RPS + rr
            acc[r, pl.ds(0, _LANES)] = 1.0 / (1.0 + jnp.exp(-v0))
            acc[r, pl.ds(_LANES, _LANES)] = 1.0 / (1.0 + jnp.exp(-v1))

    def outer(i, c):
        s0 = i * _NBUF
        for j in range(_NBUF):
            s = s0 + j
            buf = bufs[j]
            sem = sems[j]
            _wait(buf, sem)
            _step(s, buf)
            snext = s + _NBUF

            @pl.when(snext < nsteps)
            def _():
                _start(snext, buf, sem)
        return c
    lax.fori_loop(0, nsteps // _NBUF, outer, 0)

    for jt in range(ntail):
        s = nsteps - ntail + jt
        _wait(bufs[jt], sems[jt])
        _step(s, bufs[jt])

    pltpu.sync_copy(acc, out_hbm.at[pl.ds(wid * _IPT, _IPT), :])


@functools.partial(
    pl.kernel,
    mesh=plsc.VectorSubcoreMesh(core_axis_name="c", subcore_axis_name="s"),
    compiler_params=pltpu.CompilerParams(
        use_tc_tiling_on_sc=False, needs_layout_passes=False),
    out_type=jax.ShapeDtypeStruct((_B, _OUT), jnp.float32),
    scratch_types=[
        pltpu.VMEM((_L * _IPT,), jnp.int32),
        [pltpu.VMEM((_RPS * _L, _OUT), jnp.float32) for _ in range(_NBUF)],
        pltpu.VMEM((_IPT, _OUT), jnp.float32),
        pltpu.VMEM((_OUT,), jnp.float32),
        [pltpu.SemaphoreType.DMA for _ in range(_NBUF)],
    ],
)
def _pool(xr_hbm, tp_hbm, b_hbm, out_hbm,
          idx_v, bufs, acc, bias_v, sems):
    _pool_body(xr_hbm, tp_hbm, b_hbm, out_hbm,
               idx_v, bufs, acc, bias_v, sems)


def kernel(x, table, W, b):
    x = x.astype(jnp.int32)
    tp = _project_table(table, W).reshape(_VOCAB, _OUT)
    xq = (x % _VS) * _PACK + x // _VS
    xr = xq.reshape(_NW, _IPT * _L)
    return _pool(xr, tp, b)


# speedup vs baseline: 1.1192x; 1.1108x over previous
import functools

import jax
import jax.numpy as jnp
from jax import lax
from jax.experimental import pallas as pl
from jax.experimental.pallas import tpu as pltpu
from jax.experimental.pallas import tpu_sc as plsc

_VOCAB = 100000
_DIM = 128
_OUT = 32
_B = 4096
_L = 200

_NC = 2
_NS = 16
_NW = _NC * _NS
_IPT = _B // _NW
_LANES = 16


def _bf16_bits(x):
    u = lax.bitcast_convert_type(x, jnp.int32)
    return (u + jnp.int32(0x7FFF) + ((u >> 16) & 1)) >> 16


_PACK = _DIM // _OUT
_VS = _VOCAB // _PACK


def _project_body(t0, t1, t2, t3, w_ref, o_ref):
    dn = (((1,), (1,)), ((), ()))
    for j, tr in enumerate((t0, t1, t2, t3)):
        o_ref[:, _OUT * j:_OUT * (j + 1)] = lax.dot_general(
            tr[...], w_ref[...], dimension_numbers=dn,
            preferred_element_type=jnp.float32)


def _project_table(table, W):
    rows_blk = 5000
    grid = _VS // rows_blk
    in_specs = [
        pl.BlockSpec((rows_blk, _DIM),
                     (lambda g, jj=j: (g + jj * grid, 0)))
        for j in range(_PACK)
    ]
    in_specs.append(pl.BlockSpec((_OUT, _DIM), lambda g: (0, 0)))
    return pl.pallas_call(
        _project_body,
        grid=(grid,),
        in_specs=in_specs,
        out_specs=pl.BlockSpec((rows_blk, _DIM), lambda g: (g, 0)),
        out_shape=jax.ShapeDtypeStruct((_VS, _DIM), jnp.float32),
    )(table, table, table, table, W)


_NBUF = 4
_LPG = 5


def _pool_body(xr_hbm, tp_hbm, b_hbm, out_hbm,
               idx_v, bufs, acc, bias_v, sems):
    wid = lax.axis_index("s") * _NC + lax.axis_index("c")
    pltpu.sync_copy(xr_hbm.at[wid], idx_v)
    pltpu.sync_copy(b_hbm, bias_v)

    zero = jnp.zeros((_LANES,), jnp.float32)

    def zr(r, c):
        acc[r, pl.ds(0, _LANES)] = zero
        acc[r, pl.ds(_LANES, _LANES)] = zero
        return c
    lax.fori_loop(0, _IPT, zr, 0, unroll=8)

    nsteps = _L // _LPG

    for j in range(_NBUF):
        pltpu.make_async_copy(
            tp_hbm.at[idx_v.at[pl.ds(j * _LPG * _IPT, _LPG * _IPT)]],
            bufs[j], sems[j]).start()

    def outer(i, c):
        s0 = i * _NBUF
        for j in range(_NBUF):
            s = s0 + j
            buf = bufs[j]
            sem = sems[j]
            pltpu.make_async_copy(
                tp_hbm.at[idx_v.at[pl.ds(s * _LPG * _IPT, _LPG * _IPT)]],
                buf, sem).wait()

            def accum(r, cc):
                v0 = buf[r, pl.ds(0, _LANES)]
                v1 = buf[r, pl.ds(_LANES, _LANES)]
                for part in range(1, _LPG):
                    rr = part * _IPT + r
                    v0 = v0 + buf[rr, pl.ds(0, _LANES)]
                    v1 = v1 + buf[rr, pl.ds(_LANES, _LANES)]
                plsc.addupdate(acc.at[r, pl.ds(0, _LANES)], v0)
                plsc.addupdate(acc.at[r, pl.ds(_LANES, _LANES)], v1)
                return cc
            lax.fori_loop(0, _IPT, accum, 0, unroll=8)

            snext = s + _NBUF

            @pl.when(snext < nsteps)
            def _():
                pltpu.make_async_copy(
                    tp_hbm.at[idx_v.at[pl.ds(snext * _LPG * _IPT,
                                             _LPG * _IPT)]], buf, sem
                ).start()
        return c
    lax.fori_loop(0, nsteps // _NBUF, outer, 0)

    scale = jnp.float32(1.0 / _L)
    blo = bias_v[pl.ds(0, _LANES)]
    bhi = bias_v[pl.ds(_LANES, _LANES)]

    def fin(r, c):
        v0 = acc[r, pl.ds(0, _LANES)] * scale + blo
        v1 = acc[r, pl.ds(_LANES, _LANES)] * scale + bhi
        acc[r, pl.ds(0, _LANES)] = 1.0 / (1.0 + jnp.exp(-v0))
        acc[r, pl.ds(_LANES, _LANES)] = 1.0 / (1.0 + jnp.exp(-v1))
        return c
    lax.fori_loop(0, _IPT, fin, 0, unroll=4)

    pltpu.sync_copy(acc, out_hbm.at[pl.ds(wid * _IPT, _IPT), :])


@functools.partial(
    pl.kernel,
    mesh=plsc.VectorSubcoreMesh(core_axis_name="c", subcore_axis_name="s"),
    compiler_params=pltpu.CompilerParams(
        use_tc_tiling_on_sc=False, needs_layout_passes=False),
    out_type=jax.ShapeDtypeStruct((_B, _OUT), jnp.float32),
    scratch_types=[
        pltpu.VMEM((_L * _IPT,), jnp.int32),
        [pltpu.VMEM((_LPG * _IPT, _OUT), jnp.float32) for _ in range(_NBUF)],
        pltpu.VMEM((_IPT, _OUT), jnp.float32),
        pltpu.VMEM((_OUT,), jnp.float32),
        [pltpu.SemaphoreType.DMA for _ in range(_NBUF)],
    ],
)
def _pool(xr_hbm, tp_hbm, b_hbm, out_hbm,
          idx_v, bufs, acc, bias_v, sems):
    _pool_body(xr_hbm, tp_hbm, b_hbm, out_hbm,
               idx_v, bufs, acc, bias_v, sems)


def kernel(x, table, W, b):
    x = x.astype(jnp.int32)
    tp = _project_table(table, W).reshape(_VOCAB, _OUT)
    xq = (x % _VS) * _PACK + x // _VS
    xr = xq.reshape(_NW, _IPT, _L).transpose(0, 2, 1).reshape(_NW, _L * _IPT)
    return _pool(xr, tp, b)
